# baseline (speedup 1.0000x reference)
import functools

import jax
import jax.numpy as jnp
from jax import lax
from jax.experimental import pallas as pl
from jax.experimental.pallas import tpu as pltpu
from jax.experimental.pallas import tpu_sc as plsc

N_NODES = 10000
N_EDGES = 320000
NC = 2
NS = 16
CHUNK = 128
NROWS = 2560
N_EDGES_PAD = NROWS * CHUNK
ACC_ROWS = N_NODES + 128
ROWS_A = 632
ROWS_LAST = N_NODES - (NS - 1) * ROWS_A


def _degree_kernel(dst_hbm, degp_out, ones_v, dsts_v, zero_v, acc1d, semi):
  c = lax.axis_index("c")
  s = lax.axis_index("s")
  w = c * NS + s
  rpw = NROWS // (NC * NS)
  cpd = pltpu.async_copy(dst_hbm.at[pl.ds(w * rpw, rpw)], dsts_v, semi)
  for k in range(8):
    ones_v[pl.ds(k * 16, 16)] = jnp.full((16,), 1.0, jnp.float32)
  @pl.when(s == 0)
  def _():
    def zloop(i, _):
      zero_v[pl.ds(i * 16, 16)] = jnp.zeros((16,), jnp.float32)
      return 0
    lax.fori_loop(0, N_NODES // 16, zloop, 0)
    pltpu.sync_copy(zero_v, acc1d.at[pl.ds(0, N_NODES)])
  cpd.wait()
  plsc.subcore_barrier()
  ones_sl = ones_v.at[pl.ds(0, CHUNK)]
  def body(j, _):
    pltpu.sync_copy(ones_sl, acc1d.at[dsts_v.at[j]], add=True)
    return 0
  lax.fori_loop(0, rpw, body, 0)
  plsc.subcore_barrier()
  @pl.when(s == 0)
  def _():
    pltpu.sync_copy(acc1d.at[pl.ds(0, N_NODES)], zero_v)
    pltpu.sync_copy(zero_v, degp_out.at[pl.ds(c * N_NODES, N_NODES)])


@functools.cache
def _mesh():
  return plsc.VectorSubcoreMesh(core_axis_name="c", subcore_axis_name="s",
                                num_cores=NC, num_subcores=NS)


@functools.cache
def _degree():
  rpw = NROWS // (NC * NS)
  return pl.kernel(
      _degree_kernel,
      out_type=jax.ShapeDtypeStruct((NC * N_NODES,), jnp.float32),
      mesh=_mesh(),
      scratch_types=[
          pltpu.VMEM((128,), jnp.float32),
          pltpu.VMEM((rpw, CHUNK), jnp.int32),
          pltpu.VMEM((N_NODES,), jnp.float32),
          pltpu.MemorySpace.VMEM_SHARED((ACC_ROWS,), jnp.float32),
          pltpu.SemaphoreType.DMA,
      ],
      name="degree",
  )


def _propagate_kernel(h_hbm, src_hbm, dst_hbm, acc_out,
                      s0, s1, d0, d1, rows0, rows1, acc,
                      semi0, semi1, semg0, semg1, semsc0, semsc1):
  c = lax.axis_index("c")
  s = lax.axis_index("s")
  nch = NROWS // (NC * NS)
  ebase = (c * NS + s) * nch * CHUNK

  def i_start(a, sv, dv, sem):
    o = pl.multiple_of(a * CHUNK, CHUNK)
    pltpu.async_copy(src_hbm.at[pl.ds(ebase + o, CHUNK)], sv, sem)
    pltpu.async_copy(dst_hbm.at[pl.ds(ebase + o, CHUNK)], dv, sem)

  def i_wait(sv, dv, sem):
    pltpu.make_async_copy(src_hbm.at[pl.ds(ebase, CHUNK)], sv, sem).wait()
    pltpu.make_async_copy(dst_hbm.at[pl.ds(ebase, CHUNK)], dv, sem).wait()

  def g_start(sv, rows, sem):
    pltpu.async_copy(h_hbm.at[sv], rows, sem)

  def g_wait(sv, rows, sem):
    pltpu.make_async_copy(h_hbm.at[sv], rows, sem).wait()

  i_start(0, s0, d0, semi0)
  i_start(1, s1, d1, semi1)

  r0 = s * ROWS_A
  def init_io(n):
    pltpu.sync_copy(h_hbm.at[pl.ds(r0, n)], acc.at[pl.ds(r0, n)])
  @pl.when(s < NS - 1)
  def _():
    init_io(ROWS_A)
  @pl.when(s == NS - 1)
  def _():
    init_io(ROWS_LAST)
  i_wait(s0, d0, semi0)
  plsc.subcore_barrier()

  def s_start(rows, dv, sem):
    pltpu.async_copy(rows, acc.at[dv], sem, add=True)

  def s_wait(rows, dv, sem):
    pltpu.make_async_copy(rows, acc.at[dv], sem).wait()

  g_start(s0, rows0, semg0)
  def body(j, _):
    a = 2 * j
    g_wait(s0, rows0, semg0)
    i_wait(s1, d1, semi1)
    g_start(s1, rows1, semg1)
    s_start(rows0, d0, semsc0)
    g_wait(s1, rows1, semg1)
    s_start(rows1, d1, semsc1)
    s_wait(rows0, d0, semsc0)
    @pl.when(a + 2 < nch)
    def _():
      i_start(a + 2, s0, d0, semi0)
      i_wait(s0, d0, semi0)
      g_start(s0, rows0, semg0)
    s_wait(rows1, d1, semsc1)
    @pl.when(a + 3 < nch)
    def _():
      i_start(a + 3, s1, d1, semi1)
    return 0
  lax.fori_loop(0, nch // 2, body, 0)
  plsc.subcore_barrier()

  @pl.when(s < NS - 1)
  def _():
    pltpu.sync_copy(acc.at[pl.ds(r0, ROWS_A)],
                    acc_out.at[c, pl.ds(r0, ROWS_A)])
  @pl.when(s == NS - 1)
  def _():
    pltpu.sync_copy(acc.at[pl.ds(r0, ROWS_LAST)],
                    acc_out.at[c, pl.ds(r0, ROWS_LAST)])


@functools.cache
def _make_propagate():
  return pl.kernel(
      _propagate_kernel,
      out_type=jax.ShapeDtypeStruct((NC, N_NODES, 128), jnp.float32),
      mesh=_mesh(),
      scratch_types=[
          pltpu.VMEM((CHUNK,), jnp.int32),
          pltpu.VMEM((CHUNK,), jnp.int32),
          pltpu.VMEM((CHUNK,), jnp.int32),
          pltpu.VMEM((CHUNK,), jnp.int32),
          pltpu.VMEM((CHUNK, 128), jnp.float32),
          pltpu.VMEM((CHUNK, 128), jnp.float32),
          pltpu.MemorySpace.VMEM_SHARED((ACC_ROWS, 128), jnp.float32),
          pltpu.SemaphoreType.DMA,
          pltpu.SemaphoreType.DMA,
          pltpu.SemaphoreType.DMA,
          pltpu.SemaphoreType.DMA,
          pltpu.SemaphoreType.DMA,
          pltpu.SemaphoreType.DMA,
      ],
      name="propagate",
  )


_BLK = 512
_GRID = (N_NODES + _BLK - 1) // _BLK


def _dinv_from(degp):
  deg = jnp.sum(degp, axis=0) + 1.0
  return lax.rsqrt(deg)


def _tc0_kernel(x_ref, degp_ref, xs_ref):
  dinv = _dinv_from(degp_ref[...])
  xs_ref[...] = x_ref[...] * dinv[:, None]


def _tc1_kernel(acc1_ref, xs_ref, degp_ref, w1_ref, b1_ref, w2_ref, a_ref,
                h2s_ref):
  a = a_ref[0, 0]
  dinv = _dinv_from(degp_ref[...])
  agg = (acc1_ref[0] + acc1_ref[1] - xs_ref[...]) * dinv[:, None]
  h1 = jnp.dot(agg, w1_ref[...], preferred_element_type=jnp.float32)
  h1 = h1 + b1_ref[...][None, :]
  h1 = jnp.where(h1 >= 0, h1, a * h1)
  h2 = jnp.dot(h1, w2_ref[...], preferred_element_type=jnp.float32)
  h2s_ref[...] = h2 * dinv[:, None]


def _tc2_kernel(acc2_ref, h2s_ref, degp_ref, b2_ref, a_ref, out_ref):
  a = a_ref[0, 0]
  dinv = _dinv_from(degp_ref[...])
  agg = acc2_ref[0] + acc2_ref[1] - h2s_ref[...]
  out = agg * dinv[:, None] + b2_ref[...][None, :]
  out_ref[...] = jnp.where(out >= 0, out, a * out)


def _row_spec(shape_tail):
  return pl.BlockSpec((_BLK,) + shape_tail, lambda i: (i,) + (0,) * len(shape_tail))


_degp_spec = pl.BlockSpec((NC, _BLK), lambda i: (0, i))
_smem_spec = pl.BlockSpec(memory_space=pltpu.MemorySpace.SMEM)


def _full_spec(ndim):
  return pl.BlockSpec(None, lambda i: (0,) * ndim)


_acc_spec = pl.BlockSpec((NC, _BLK, 128), lambda i: (0, i, 0))

_tc0 = pl.pallas_call(
    _tc0_kernel,
    grid=(_GRID,),
    in_specs=[_row_spec((128,)), _degp_spec],
    out_specs=_row_spec((128,)),
    out_shape=jax.ShapeDtypeStruct((N_NODES, 128), jnp.float32),
)

_tc1 = pl.pallas_call(
    _tc1_kernel,
    grid=(_GRID,),
    in_specs=[_acc_spec, _row_spec((128,)), _degp_spec,
              _full_spec(2), _full_spec(1), _full_spec(2), _smem_spec],
    out_specs=_row_spec((128,)),
    out_shape=jax.ShapeDtypeStruct((N_NODES, 128), jnp.float32),
)

_tc2 = pl.pallas_call(
    _tc2_kernel,
    grid=(_GRID,),
    in_specs=[_acc_spec, _row_spec((128,)), _degp_spec,
              _full_spec(1), _smem_spec],
    out_specs=_row_spec((128,)),
    out_shape=jax.ShapeDtypeStruct((N_NODES, 128), jnp.float32),
)


@jax.jit
def kernel(x, edge_index, W1, b1, W2, b2, prelu_a):
  npad = N_EDGES_PAD - N_EDGES
  src = jnp.concatenate(
      [edge_index[0].astype(jnp.int32),
       jnp.arange(npad, dtype=jnp.int32) % N_NODES])
  dst = jnp.concatenate(
      [edge_index[1].astype(jnp.int32),
       N_NODES + (jnp.arange(npad, dtype=jnp.int32) % 128)])
  a = jnp.reshape(prelu_a.astype(jnp.float32), (1, 1))

  prop = _make_propagate()
  degp = _degree()(dst.reshape(NROWS, CHUNK)).reshape(NC, N_NODES)
  xs = _tc0(x, degp)
  acc1 = prop(xs, src, dst)
  h2s = _tc1(acc1, xs, degp, W1, b1, W2, a)
  acc2 = prop(h2s, src, dst)
  return _tc2(acc2, h2s, degp, b2, a)


# speedup vs baseline: 1.0057x; 1.0057x over previous
import functools

import jax
import jax.numpy as jnp
from jax import lax
from jax.experimental import pallas as pl
from jax.experimental.pallas import tpu as pltpu
from jax.experimental.pallas import tpu_sc as plsc

N_NODES = 10000
N_EDGES = 320000
NC = 2
NS = 16
CHUNK = 128
NROWS = 2560
N_EDGES_PAD = NROWS * CHUNK
ACC_ROWS = N_NODES + 128
ROWS_A = 632
ROWS_LAST = N_NODES - (NS - 1) * ROWS_A


def _degree_kernel(dst_hbm, degp_out, ones_v, dsts_v, zero_v, acc1d, semi,
                   semd0, semd1):
  c = lax.axis_index("c")
  s = lax.axis_index("s")
  w = c * NS + s
  rpw = NROWS // (NC * NS)
  cpd = pltpu.async_copy(dst_hbm.at[pl.ds(w * rpw, rpw)], dsts_v, semi)
  for k in range(8):
    ones_v[pl.ds(k * 16, 16)] = jnp.full((16,), 1.0, jnp.float32)
  @pl.when(s == 0)
  def _():
    def zloop(i, _):
      zero_v[pl.ds(i * 16, 16)] = jnp.zeros((16,), jnp.float32)
      return 0
    lax.fori_loop(0, N_NODES // 16, zloop, 0)
    pltpu.sync_copy(zero_v, acc1d.at[pl.ds(0, N_NODES)])
  cpd.wait()
  plsc.subcore_barrier()
  ones_sl = ones_v.at[pl.ds(0, CHUNK)]
  def d_start(j, sem):
    pltpu.async_copy(ones_sl, acc1d.at[dsts_v.at[j]], sem, add=True)
  def d_wait(j, sem):
    pltpu.make_async_copy(ones_sl, acc1d.at[dsts_v.at[j]], sem).wait()
  d_start(0, semd0)
  def body(jj, _):
    a = 2 * jj
    d_start(a + 1, semd1)
    d_wait(a, semd0)
    @pl.when(a + 2 < rpw)
    def _():
      d_start(a + 2, semd0)
    d_wait(a + 1, semd1)
    return 0
  lax.fori_loop(0, rpw // 2, body, 0)
  plsc.subcore_barrier()
  @pl.when(s == 0)
  def _():
    pltpu.sync_copy(acc1d.at[pl.ds(0, N_NODES)], zero_v)
    pltpu.sync_copy(zero_v, degp_out.at[pl.ds(c * N_NODES, N_NODES)])


@functools.cache
def _mesh():
  return plsc.VectorSubcoreMesh(core_axis_name="c", subcore_axis_name="s",
                                num_cores=NC, num_subcores=NS)


@functools.cache
def _degree():
  rpw = NROWS // (NC * NS)
  return pl.kernel(
      _degree_kernel,
      out_type=jax.ShapeDtypeStruct((NC * N_NODES,), jnp.float32),
      mesh=_mesh(),
      scratch_types=[
          pltpu.VMEM((128,), jnp.float32),
          pltpu.VMEM((rpw, CHUNK), jnp.int32),
          pltpu.VMEM((N_NODES,), jnp.float32),
          pltpu.MemorySpace.VMEM_SHARED((ACC_ROWS,), jnp.float32),
          pltpu.SemaphoreType.DMA,
          pltpu.SemaphoreType.DMA,
          pltpu.SemaphoreType.DMA,
      ],
      name="degree",
  )


def _propagate_kernel(h_hbm, src_hbm, dst_hbm, acc_out,
                      s0, s1, d0, d1, rows0, rows1, acc,
                      semi0, semi1, semg0, semg1, semsc0, semsc1):
  c = lax.axis_index("c")
  s = lax.axis_index("s")
  nch = NROWS // (NC * NS)
  ebase = (c * NS + s) * nch * CHUNK

  def i_start(a, sv, dv, sem):
    o = pl.multiple_of(a * CHUNK, CHUNK)
    pltpu.async_copy(src_hbm.at[pl.ds(ebase + o, CHUNK)], sv, sem)
    pltpu.async_copy(dst_hbm.at[pl.ds(ebase + o, CHUNK)], dv, sem)

  def i_wait(sv, dv, sem):
    pltpu.make_async_copy(src_hbm.at[pl.ds(ebase, CHUNK)], sv, sem).wait()
    pltpu.make_async_copy(dst_hbm.at[pl.ds(ebase, CHUNK)], dv, sem).wait()

  def g_start(sv, rows, sem):
    pltpu.async_copy(h_hbm.at[sv], rows, sem)

  def g_wait(sv, rows, sem):
    pltpu.make_async_copy(h_hbm.at[sv], rows, sem).wait()

  i_start(0, s0, d0, semi0)
  i_start(1, s1, d1, semi1)

  r0 = s * ROWS_A
  def init_io(n):
    pltpu.sync_copy(h_hbm.at[pl.ds(r0, n)], acc.at[pl.ds(r0, n)])
  @pl.when(s < NS - 1)
  def _():
    init_io(ROWS_A)
  @pl.when(s == NS - 1)
  def _():
    init_io(ROWS_LAST)
  i_wait(s0, d0, semi0)
  plsc.subcore_barrier()

  def s_start(rows, dv, sem):
    pltpu.async_copy(rows, acc.at[dv], sem, add=True)

  def s_wait(rows, dv, sem):
    pltpu.make_async_copy(rows, acc.at[dv], sem).wait()

  g_start(s0, rows0, semg0)
  def body(j, _):
    a = 2 * j
    g_wait(s0, rows0, semg0)
    i_wait(s1, d1, semi1)
    g_start(s1, rows1, semg1)
    s_start(rows0, d0, semsc0)
    g_wait(s1, rows1, semg1)
    s_start(rows1, d1, semsc1)
    s_wait(rows0, d0, semsc0)
    @pl.when(a + 2 < nch)
    def _():
      i_start(a + 2, s0, d0, semi0)
      i_wait(s0, d0, semi0)
      g_start(s0, rows0, semg0)
    s_wait(rows1, d1, semsc1)
    @pl.when(a + 3 < nch)
    def _():
      i_start(a + 3, s1, d1, semi1)
    return 0
  lax.fori_loop(0, nch // 2, body, 0)
  plsc.subcore_barrier()

  @pl.when(s < NS - 1)
  def _():
    pltpu.sync_copy(acc.at[pl.ds(r0, ROWS_A)],
                    acc_out.at[c, pl.ds(r0, ROWS_A)])
  @pl.when(s == NS - 1)
  def _():
    pltpu.sync_copy(acc.at[pl.ds(r0, ROWS_LAST)],
                    acc_out.at[c, pl.ds(r0, ROWS_LAST)])


@functools.cache
def _make_propagate():
  return pl.kernel(
      _propagate_kernel,
      out_type=jax.ShapeDtypeStruct((NC, N_NODES, 128), jnp.float32),
      mesh=_mesh(),
      scratch_types=[
          pltpu.VMEM((CHUNK,), jnp.int32),
          pltpu.VMEM((CHUNK,), jnp.int32),
          pltpu.VMEM((CHUNK,), jnp.int32),
          pltpu.VMEM((CHUNK,), jnp.int32),
          pltpu.VMEM((CHUNK, 128), jnp.float32),
          pltpu.VMEM((CHUNK, 128), jnp.float32),
          pltpu.MemorySpace.VMEM_SHARED((ACC_ROWS, 128), jnp.float32),
          pltpu.SemaphoreType.DMA,
          pltpu.SemaphoreType.DMA,
          pltpu.SemaphoreType.DMA,
          pltpu.SemaphoreType.DMA,
          pltpu.SemaphoreType.DMA,
          pltpu.SemaphoreType.DMA,
      ],
      name="propagate",
  )


_BLK = 512
_GRID = (N_NODES + _BLK - 1) // _BLK


def _dinv_from(degp):
  deg = jnp.sum(degp, axis=0) + 1.0
  return lax.rsqrt(deg)


def _tc0_kernel(x_ref, degp_ref, xs_ref):
  dinv = _dinv_from(degp_ref[...])
  xs_ref[...] = x_ref[...] * dinv[:, None]


def _tc1_kernel(acc1_ref, xs_ref, degp_ref, w1_ref, b1_ref, w2_ref, a_ref,
                h2s_ref):
  a = a_ref[0, 0]
  dinv = _dinv_from(degp_ref[...])
  agg = (acc1_ref[0] + acc1_ref[1] - xs_ref[...]) * dinv[:, None]
  h1 = jnp.dot(agg, w1_ref[...], preferred_element_type=jnp.float32)
  h1 = h1 + b1_ref[...][None, :]
  h1 = jnp.where(h1 >= 0, h1, a * h1)
  h2 = jnp.dot(h1, w2_ref[...], preferred_element_type=jnp.float32)
  h2s_ref[...] = h2 * dinv[:, None]


def _tc2_kernel(acc2_ref, h2s_ref, degp_ref, b2_ref, a_ref, out_ref):
  a = a_ref[0, 0]
  dinv = _dinv_from(degp_ref[...])
  agg = acc2_ref[0] + acc2_ref[1] - h2s_ref[...]
  out = agg * dinv[:, None] + b2_ref[...][None, :]
  out_ref[...] = jnp.where(out >= 0, out, a * out)


def _row_spec(shape_tail):
  return pl.BlockSpec((_BLK,) + shape_tail, lambda i: (i,) + (0,) * len(shape_tail))


_degp_spec = pl.BlockSpec((NC, _BLK), lambda i: (0, i))
_smem_spec = pl.BlockSpec(memory_space=pltpu.MemorySpace.SMEM)


def _full_spec(ndim):
  return pl.BlockSpec(None, lambda i: (0,) * ndim)


_acc_spec = pl.BlockSpec((NC, _BLK, 128), lambda i: (0, i, 0))

_tc0 = pl.pallas_call(
    _tc0_kernel,
    grid=(_GRID,),
    in_specs=[_row_spec((128,)), _degp_spec],
    out_specs=_row_spec((128,)),
    out_shape=jax.ShapeDtypeStruct((N_NODES, 128), jnp.float32),
)

_tc1 = pl.pallas_call(
    _tc1_kernel,
    grid=(_GRID,),
    in_specs=[_acc_spec, _row_spec((128,)), _degp_spec,
              _full_spec(2), _full_spec(1), _full_spec(2), _smem_spec],
    out_specs=_row_spec((128,)),
    out_shape=jax.ShapeDtypeStruct((N_NODES, 128), jnp.float32),
)

_tc2 = pl.pallas_call(
    _tc2_kernel,
    grid=(_GRID,),
    in_specs=[_acc_spec, _row_spec((128,)), _degp_spec,
              _full_spec(1), _smem_spec],
    out_specs=_row_spec((128,)),
    out_shape=jax.ShapeDtypeStruct((N_NODES, 128), jnp.float32),
)


@jax.jit
def kernel(x, edge_index, W1, b1, W2, b2, prelu_a):
  npad = N_EDGES_PAD - N_EDGES
  src = jnp.concatenate(
      [edge_index[0].astype(jnp.int32),
       jnp.arange(npad, dtype=jnp.int32) % N_NODES])
  dst = jnp.concatenate(
      [edge_index[1].astype(jnp.int32),
       N_NODES + (jnp.arange(npad, dtype=jnp.int32) % 128)])
  a = jnp.reshape(prelu_a.astype(jnp.float32), (1, 1))

  prop = _make_propagate()
  degp = _degree()(dst.reshape(NROWS, CHUNK)).reshape(NC, N_NODES)
  xs = _tc0(x, degp)
  acc1 = prop(xs, src, dst)
  h2s = _tc1(acc1, xs, degp, W1, b1, W2, a)
  acc2 = prop(h2s, src, dst)
  return _tc2(acc2, h2s, degp, b2, a)
